# baseline (speedup 1.0000x reference)
import functools

import jax
import jax.numpy as jnp
from jax import lax
from jax.experimental import pallas as pl
from jax.experimental.pallas import tpu as pltpu
from jax.experimental.pallas import tpu_sc as plsc

EMB = 32
CHUNK = 128
NUM_WORKERS = 32
L = 16


@functools.cache
def _build(n_rows, n_fields):
    rows_per_w = n_rows // NUM_WORKERS
    chunks_per_f = rows_per_w // CHUNK
    n_chunks = n_fields * chunks_per_f
    mesh = plsc.VectorSubcoreMesh(core_axis_name="c", subcore_axis_name="s")

    @functools.partial(
        pl.kernel,
        mesh=mesh,
        out_type=jax.ShapeDtypeStruct(
            (n_fields, EMB // 8, n_rows // 128, 8, 128), jnp.float32
        ),
        scratch_types=[
            pltpu.VMEM((rows_per_w, n_fields), jnp.int32),
            pltpu.VMEM((n_fields, rows_per_w), jnp.int32),
            pltpu.VMEM((2, CHUNK, EMB), jnp.float32),
            pltpu.VMEM((2, EMB // 8, 8, CHUNK), jnp.float32),
            pltpu.SemaphoreType.DMA((2,)),
            pltpu.SemaphoreType.DMA((2,)),
        ],
        compiler_params=pltpu.CompilerParams(use_tc_tiling_on_sc=False, needs_layout_passes=False),
    )
    def gather_kernel(idx_hbm, table_hbm, out_hbm, idx_v, idxt_v, rows_v, rowst_v,
                      gsem, osem):
        iota = lax.iota(jnp.int32, L)
        wid = lax.axis_index("s") * 2 + lax.axis_index("c")
        base_row = wid * rows_per_w
        base_chunk = base_row // CHUNK
        pltpu.sync_copy(idx_hbm.at[pl.ds(base_row, rows_per_w)], idx_v)

        blocks_per_f = rows_per_w // L

        for j in range(n_fields):
            cols = jnp.full((L,), j, jnp.int32)

            def repack(blk, carry, j=j, cols=cols):
                rows = blk * L + iota
                idxt_v[j, pl.ds(blk * L, L)] = plsc.load_gather(idx_v, [rows, cols])
                return carry

            lax.fori_loop(0, blocks_per_f, repack, 0)

        def fire_gather(t, b):
            j = t // chunks_per_f
            c = t % chunks_per_f
            pltpu.async_copy(
                table_hbm.at[idxt_v.at[j, pl.ds(c * CHUNK, CHUNK)]],
                rows_v.at[b],
                gsem.at[b],
            )

        def drain_gather(b):
            pltpu.make_async_copy(
                table_hbm.at[idxt_v.at[0, pl.ds(0, CHUNK)]],
                rows_v.at[b],
                gsem.at[b],
            ).wait()

        def transpose_chunk(b):
            for e in range(EMB):
                cols = jnp.zeros((L,), jnp.int32) + e
                for blk in range(CHUNK // L):
                    rows = blk * L + iota
                    rowst_v[b, e // 8, e % 8, pl.ds(blk * L, L)] = plsc.load_gather(
                        rows_v.at[b], [rows, cols]
                    )

        def fire_store(t, b):
            j = t // chunks_per_f
            c = t % chunks_per_f
            pltpu.async_copy(
                rowst_v.at[b],
                out_hbm.at[j, :, base_chunk + c],
                osem.at[b],
            )

        def drain_store(b):
            pltpu.make_async_copy(
                rowst_v.at[b],
                out_hbm.at[0, :, 0],
                osem.at[b],
            ).wait()

        fire_gather(0, 0)

        def body(p, carry):
            for b in range(2):
                t = 2 * p + b

                @pl.when(t + 1 < n_chunks)
                def _(t=t, b=b):
                    fire_gather(t + 1, 1 - b)

                drain_gather(b)

                @pl.when(t >= 2)
                def _(b=b):
                    drain_store(b)

                transpose_chunk(b)
                fire_store(t, b)
            return carry

        lax.fori_loop(0, n_chunks // 2, body, 0)
        drain_store(0)
        drain_store(1)

    return gather_kernel


def kernel(indices, table):
    n_rows, n_fields = indices.shape
    out5d = _build(n_rows, n_fields)(indices.astype(jnp.int32), table)
    return out5d.transpose(2, 4, 0, 1, 3).reshape(n_rows, n_fields, EMB)


# speedup vs baseline: 1.1559x; 1.1559x over previous
import functools

import jax
import jax.numpy as jnp
from jax import lax
from jax.experimental import pallas as pl
from jax.experimental.pallas import tpu as pltpu
from jax.experimental.pallas import tpu_sc as plsc

EMB = 32
CHUNK = 128
NUM_WORKERS = 32
L = 16


@functools.cache
def _build(n_rows, n_fields):
    rows_per_w = n_rows // NUM_WORKERS
    chunks_per_f = rows_per_w // CHUNK
    n_chunks = n_fields * chunks_per_f
    blocks_per_f = rows_per_w // L
    mesh = plsc.VectorSubcoreMesh(core_axis_name="c", subcore_axis_name="s")

    @functools.partial(
        pl.kernel,
        mesh=mesh,
        out_type=jax.ShapeDtypeStruct(
            (n_fields, EMB // 8, n_rows // 128, 8 * 128), jnp.float32
        ),
        scratch_types=[
            pltpu.VMEM((rows_per_w, n_fields), jnp.int32),
            pltpu.VMEM((rows_per_w * n_fields,), jnp.int32),
            pltpu.VMEM((2, CHUNK, EMB), jnp.float32),
            pltpu.VMEM((2, EMB * CHUNK), jnp.float32),
            pltpu.SemaphoreType.DMA((2,)),
            pltpu.SemaphoreType.DMA((2,)),
        ],
        compiler_params=pltpu.CompilerParams(
            use_tc_tiling_on_sc=False, needs_layout_passes=False
        ),
    )
    def gather_kernel(idx_hbm, table_hbm, out_hbm, idx_v, idxt_v, rows_v, rowst_v,
                      gsem, osem):
        iota = lax.iota(jnp.int32, L)
        zeros = jnp.zeros((L,), jnp.int32)
        wid = lax.axis_index("s") * 2 + lax.axis_index("c")
        base_row = wid * rows_per_w
        base_chunk = base_row // CHUNK
        pltpu.sync_copy(idx_hbm.at[pl.ds(base_row, rows_per_w)], idx_v)

        @plsc.parallel_loop(0, n_fields * blocks_per_f, unroll=8)
        def _repack(q):
            j = q // blocks_per_f
            blk = q % blocks_per_f
            vals = plsc.load_gather(idx_v, [blk * L + iota, zeros + j])
            idxt_v[pl.ds(q * L, L)] = vals

        def fire_gather(t, b):
            pltpu.async_copy(
                table_hbm.at[idxt_v.at[pl.ds(t * CHUNK, CHUNK)]],
                rows_v.at[b],
                gsem.at[b],
            )

        def drain_gather(b):
            pltpu.make_async_copy(
                table_hbm.at[idxt_v.at[pl.ds(0, CHUNK)]],
                rows_v.at[b],
                gsem.at[b],
            ).wait()

        def transpose_chunk(b):
            @plsc.parallel_loop(0, EMB * (CHUNK // L), unroll=8)
            def _tq(q):
                e = q // (CHUNK // L)
                blk = q % (CHUNK // L)
                vals = plsc.load_gather(rows_v.at[b], [blk * L + iota, zeros + e])
                rowst_v[b, pl.ds(e * CHUNK + blk * L, L)] = vals

        def fire_store(t, b):
            j = t // chunks_per_f
            c = t % chunks_per_f
            for ec in range(EMB // 8):
                pltpu.async_copy(
                    rowst_v.at[b, pl.ds(ec * 8 * CHUNK, 8 * CHUNK)],
                    out_hbm.at[j, ec, base_chunk + c],
                    osem.at[b],
                )

        def drain_store(b):
            for ec in range(EMB // 8):
                pltpu.make_async_copy(
                    rowst_v.at[b, pl.ds(ec * 8 * CHUNK, 8 * CHUNK)],
                    out_hbm.at[0, ec, 0],
                    osem.at[b],
                ).wait()

        fire_gather(0, 0)

        def body(p, carry):
            for b in range(2):
                t = 2 * p + b

                @pl.when(t + 1 < n_chunks)
                def _(t=t, b=b):
                    fire_gather(t + 1, 1 - b)

                drain_gather(b)

                @pl.when(t >= 2)
                def _(b=b):
                    drain_store(b)

                transpose_chunk(b)
                fire_store(t, b)
            return carry

        lax.fori_loop(0, n_chunks // 2, body, 0)
        drain_store(0)
        drain_store(1)

    return gather_kernel


def kernel(indices, table):
    n_rows, n_fields = indices.shape
    out4d = _build(n_rows, n_fields)(indices.astype(jnp.int32), table)
    out5d = out4d.reshape(n_fields, EMB // 8, n_rows // 128, 8, 128)
    return out5d.transpose(2, 4, 0, 1, 3).reshape(n_rows, n_fields, EMB)


# speedup vs baseline: 1.4999x; 1.2975x over previous
import functools

import jax
import jax.numpy as jnp
from jax import lax
from jax.experimental import pallas as pl
from jax.experimental.pallas import tpu as pltpu
from jax.experimental.pallas import tpu_sc as plsc

EMB = 32
CHUNK = 128
NUM_WORKERS = 32
L = 16


@functools.cache
def _build(n_rows, n_fields):
    rows_per_w = n_rows // NUM_WORKERS
    chunks_per_f = rows_per_w // CHUNK
    n_chunks = n_fields * chunks_per_f
    blocks_per_f = rows_per_w // L
    mesh = plsc.VectorSubcoreMesh(core_axis_name="c", subcore_axis_name="s")

    @functools.partial(
        pl.kernel,
        mesh=mesh,
        out_type=jax.ShapeDtypeStruct(
            (n_fields, EMB // 8, n_rows // 128, 8, 128), jnp.float32
        ),
        scratch_types=[
            pltpu.VMEM((rows_per_w, n_fields), jnp.int32),
            pltpu.VMEM((rows_per_w * n_fields,), jnp.int32),
            pltpu.VMEM((2, CHUNK, EMB), jnp.float32),
            pltpu.VMEM((2, EMB // 8, 8, CHUNK + 1), jnp.float32),
            pltpu.SemaphoreType.DMA((2,)),
            pltpu.SemaphoreType.DMA((2,)),
        ],
        compiler_params=pltpu.CompilerParams(
            use_tc_tiling_on_sc=False, needs_layout_passes=False
        ),
    )
    def gather_kernel(idx_hbm, table_hbm, out_hbm, idx_v, idxt_v, rows_v, rowst_v,
                      gsem, osem):
        iota = lax.iota(jnp.int32, L)
        zeros = jnp.zeros((L,), jnp.int32)
        wid = lax.axis_index("s") * 2 + lax.axis_index("c")
        base_row = wid * rows_per_w
        base_chunk = base_row // CHUNK
        pltpu.sync_copy(idx_hbm.at[pl.ds(base_row, rows_per_w)], idx_v)

        @plsc.parallel_loop(0, n_fields * blocks_per_f, unroll=8)
        def _repack(q):
            j = q // blocks_per_f
            blk = q % blocks_per_f
            vals = plsc.load_gather(idx_v, [blk * L + iota, zeros + j])
            idxt_v[pl.ds(q * L, L)] = vals

        def fire_gather(t, b):
            pltpu.async_copy(
                table_hbm.at[idxt_v.at[pl.ds(t * CHUNK, CHUNK)]],
                rows_v.at[b],
                gsem.at[b],
            )

        def drain_gather(b):
            pltpu.make_async_copy(
                table_hbm.at[idxt_v.at[pl.ds(0, CHUNK)]],
                rows_v.at[b],
                gsem.at[b],
            ).wait()

        ec1, el1 = iota // 8, iota % 8
        ec2, el2 = (iota + L) // 8, (iota + L) % 8

        def transpose_chunk(b):
            @plsc.parallel_loop(0, CHUNK, unroll=8, carry=zeros)
            def _ti(i, civ):
                plsc.store_scatter(
                    rowst_v.at[b], [ec1, el1, civ], rows_v[b, i, pl.ds(0, L)]
                )
                plsc.store_scatter(
                    rowst_v.at[b], [ec2, el2, civ], rows_v[b, i, pl.ds(L, L)]
                )
                return civ + 1

        def fire_store(t, b):
            j = t // chunks_per_f
            c = t % chunks_per_f
            pltpu.async_copy(
                rowst_v.at[b, :, :, pl.ds(0, CHUNK)],
                out_hbm.at[j, :, base_chunk + c],
                osem.at[b],
            )

        def drain_store(b):
            pltpu.make_async_copy(
                rowst_v.at[b, :, :, pl.ds(0, CHUNK)],
                out_hbm.at[0, :, 0],
                osem.at[b],
            ).wait()

        fire_gather(0, 0)

        def body(p, carry):
            for b in range(2):
                t = 2 * p + b

                @pl.when(t + 1 < n_chunks)
                def _(t=t, b=b):
                    fire_gather(t + 1, 1 - b)

                drain_gather(b)

                @pl.when(t >= 2)
                def _(b=b):
                    drain_store(b)

                transpose_chunk(b)
                fire_store(t, b)
            return carry

        lax.fori_loop(0, n_chunks // 2, body, 0)
        drain_store(0)
        drain_store(1)

    return gather_kernel


def kernel(indices, table):
    n_rows, n_fields = indices.shape
    out5d = _build(n_rows, n_fields)(indices.astype(jnp.int32), table)
    return out5d.transpose(2, 4, 0, 1, 3).reshape(n_rows, n_fields, EMB)
